# baseline (speedup 1.0000x reference)
import functools

import jax
import jax.numpy as jnp
from jax.experimental import pallas as pl
from jax.experimental.pallas import tpu as pltpu

_EPS = 1e-8
_BLOCK_K = 2000


def _grace_kernel(query_ref, keys_ref, dists_ref, nearest_ref, smallest_ref,
                  qn_ref, minv_ref, mini_ref, *, block_k, nsteps):
    i = pl.program_id(0)

    @pl.when(i == 0)
    def _init():
        q = query_ref[:]
        qnorm = jnp.sqrt(jnp.sum(q * q, axis=1, keepdims=True))
        qn_ref[:] = q / jnp.maximum(qnorm, _EPS)
        minv_ref[:] = jnp.full(minv_ref.shape, jnp.inf, jnp.float32)
        mini_ref[:] = jnp.zeros(mini_ref.shape, jnp.int32)

    k = keys_ref[:]
    knorm = jnp.sqrt(jnp.sum(k * k, axis=1, keepdims=True))
    inv = 1.0 / jnp.maximum(knorm, _EPS)
    sim = jax.lax.dot_general(
        k, qn_ref[:], (((1,), (1,)), ((), ())),
        preferred_element_type=jnp.float32,
        precision=jax.lax.Precision.HIGHEST)
    dist = 1.0 - sim * inv
    dists_ref[:] = dist

    bmin = jnp.min(dist, axis=0)
    rows = jax.lax.broadcasted_iota(jnp.int32, dist.shape, 0)
    masked = jnp.where(dist == bmin[None, :], rows, block_k)
    barg = jnp.min(masked, axis=0) + i * block_k

    run_min = minv_ref[0, :]
    run_arg = mini_ref[0, :]
    better = bmin < run_min
    minv_ref[0, :] = jnp.where(better, bmin, run_min)
    mini_ref[0, :] = jnp.where(better, barg, run_arg)

    @pl.when(i == nsteps - 1)
    def _fin():
        smallest_ref[:] = minv_ref[0, :]
        nearest_ref[:] = mini_ref[0, :]


@jax.jit
def kernel(query, keys):
    num_keys, d = keys.shape
    q = query.shape[0]
    block_k = _BLOCK_K
    nsteps = num_keys // block_k
    dists, nearest, smallest = pl.pallas_call(
        functools.partial(_grace_kernel, block_k=block_k, nsteps=nsteps),
        grid=(nsteps,),
        in_specs=[
            pl.BlockSpec((q, d), lambda i: (0, 0)),
            pl.BlockSpec((block_k, d), lambda i: (i, 0)),
        ],
        out_specs=[
            pl.BlockSpec((block_k, q), lambda i: (i, 0)),
            pl.BlockSpec((q,), lambda i: (0,)),
            pl.BlockSpec((q,), lambda i: (0,)),
        ],
        out_shape=[
            jax.ShapeDtypeStruct((num_keys, q), jnp.float32),
            jax.ShapeDtypeStruct((q,), jnp.int32),
            jax.ShapeDtypeStruct((q,), jnp.float32),
        ],
        scratch_shapes=[
            pltpu.VMEM((q, d), jnp.float32),
            pltpu.VMEM((1, q), jnp.float32),
            pltpu.VMEM((1, q), jnp.int32),
        ],
    )(query, keys)
    return dists, nearest, smallest


# speedup vs baseline: 2.3015x; 2.3015x over previous
import functools

import jax
import jax.numpy as jnp
from jax.experimental import pallas as pl
from jax.experimental.pallas import tpu as pltpu

_EPS = 1e-8
_BLOCK_K = 2000


def _grace_kernel(query_ref, keys_ref, dists_ref, nearest_ref, smallest_ref,
                  qn_ref, minv_ref, mini_ref, *, block_k, nsteps):
    i = pl.program_id(0)

    @pl.when(i == 0)
    def _init():
        q = query_ref[:]
        qnorm = jnp.sqrt(jnp.sum(q * q, axis=1, keepdims=True))
        qn = q / jnp.maximum(qnorm, _EPS)
        qn_ref[:] = qn.astype(jnp.bfloat16)
        minv_ref[:] = jnp.full(minv_ref.shape, jnp.inf, jnp.float32)
        mini_ref[:] = jnp.zeros(mini_ref.shape, jnp.int32)

    k = keys_ref[:]
    knorm = jnp.sqrt(jnp.sum(k * k, axis=1, keepdims=True))
    kn = (k / jnp.maximum(knorm, _EPS)).astype(jnp.bfloat16)
    sim = jax.lax.dot_general(
        kn, qn_ref[:], (((1,), (1,)), ((), ())),
        preferred_element_type=jnp.float32)
    dist = 1.0 - sim
    dists_ref[:] = dist

    bmin = jnp.min(dist, axis=0)
    rows = jax.lax.broadcasted_iota(jnp.int32, dist.shape, 0)
    masked = jnp.where(dist == bmin[None, :], rows, block_k)
    barg = jnp.min(masked, axis=0) + i * block_k

    run_min = minv_ref[0, :]
    run_arg = mini_ref[0, :]
    better = bmin < run_min
    minv_ref[0, :] = jnp.where(better, bmin, run_min)
    mini_ref[0, :] = jnp.where(better, barg, run_arg)

    @pl.when(i == nsteps - 1)
    def _fin():
        smallest_ref[:] = minv_ref[0, :]
        nearest_ref[:] = mini_ref[0, :]


@jax.jit
def kernel(query, keys):
    num_keys, d = keys.shape
    q = query.shape[0]
    block_k = _BLOCK_K
    nsteps = num_keys // block_k
    dists, nearest, smallest = pl.pallas_call(
        functools.partial(_grace_kernel, block_k=block_k, nsteps=nsteps),
        grid=(nsteps,),
        in_specs=[
            pl.BlockSpec((q, d), lambda i: (0, 0)),
            pl.BlockSpec((block_k, d), lambda i: (i, 0)),
        ],
        out_specs=[
            pl.BlockSpec((block_k, q), lambda i: (i, 0)),
            pl.BlockSpec((q,), lambda i: (0,)),
            pl.BlockSpec((q,), lambda i: (0,)),
        ],
        out_shape=[
            jax.ShapeDtypeStruct((num_keys, q), jnp.float32),
            jax.ShapeDtypeStruct((q,), jnp.int32),
            jax.ShapeDtypeStruct((q,), jnp.float32),
        ],
        scratch_shapes=[
            pltpu.VMEM((q, d), jnp.bfloat16),
            pltpu.VMEM((1, q), jnp.float32),
            pltpu.VMEM((1, q), jnp.int32),
        ],
    )(query, keys)
    return dists, nearest, smallest


# speedup vs baseline: 2.4473x; 1.0634x over previous
import functools

import jax
import jax.numpy as jnp
from jax.experimental import pallas as pl
from jax.experimental.pallas import tpu as pltpu

_EPS = 1e-8
_BLOCK_K = 4000


def _grace_kernel(query_ref, keys_ref, dists_ref, nearest_ref, smallest_ref,
                  qn_ref, minv_ref, mini_ref, *, block_k, nsteps):
    i = pl.program_id(0)

    @pl.when(i == 0)
    def _init():
        q = query_ref[:]
        qnorm = jnp.sqrt(jnp.sum(q * q, axis=1, keepdims=True))
        qn = q / jnp.maximum(qnorm, _EPS)
        qn_ref[:] = qn.astype(jnp.bfloat16)
        minv_ref[:] = jnp.full(minv_ref.shape, jnp.inf, jnp.float32)
        mini_ref[:] = jnp.zeros(mini_ref.shape, jnp.int32)

    k = keys_ref[:]
    knorm = jnp.sqrt(jnp.sum(k * k, axis=1, keepdims=True))
    kn = (k / jnp.maximum(knorm, _EPS)).astype(jnp.bfloat16)
    sim = jax.lax.dot_general(
        kn, qn_ref[:], (((1,), (1,)), ((), ())),
        preferred_element_type=jnp.float32)
    dist = 1.0 - sim
    dists_ref[:] = dist

    bmin = jnp.min(dist, axis=0)
    rows = jax.lax.broadcasted_iota(jnp.int32, dist.shape, 0)
    masked = jnp.where(dist == bmin[None, :], rows, block_k)
    barg = jnp.min(masked, axis=0) + i * block_k

    run_min = minv_ref[0, :]
    run_arg = mini_ref[0, :]
    better = bmin < run_min
    minv_ref[0, :] = jnp.where(better, bmin, run_min)
    mini_ref[0, :] = jnp.where(better, barg, run_arg)

    @pl.when(i == nsteps - 1)
    def _fin():
        smallest_ref[:] = minv_ref[0, :]
        nearest_ref[:] = mini_ref[0, :]


@jax.jit
def kernel(query, keys):
    num_keys, d = keys.shape
    q = query.shape[0]
    block_k = _BLOCK_K
    nsteps = num_keys // block_k
    dists, nearest, smallest = pl.pallas_call(
        functools.partial(_grace_kernel, block_k=block_k, nsteps=nsteps),
        grid=(nsteps,),
        in_specs=[
            pl.BlockSpec((q, d), lambda i: (0, 0)),
            pl.BlockSpec((block_k, d), lambda i: (i, 0)),
        ],
        out_specs=[
            pl.BlockSpec((block_k, q), lambda i: (i, 0)),
            pl.BlockSpec((q,), lambda i: (0,)),
            pl.BlockSpec((q,), lambda i: (0,)),
        ],
        out_shape=[
            jax.ShapeDtypeStruct((num_keys, q), jnp.float32),
            jax.ShapeDtypeStruct((q,), jnp.int32),
            jax.ShapeDtypeStruct((q,), jnp.float32),
        ],
        scratch_shapes=[
            pltpu.VMEM((q, d), jnp.bfloat16),
            pltpu.VMEM((1, q), jnp.float32),
            pltpu.VMEM((1, q), jnp.int32),
        ],
    )(query, keys)
    return dists, nearest, smallest
